# baseline (speedup 1.0000x reference)
import functools
import jax
import jax.numpy as jnp
from jax import lax
from jax.experimental import pallas as pl
from jax.experimental.pallas import tpu as pltpu, tpu_sc as plsc

B, S, H, D = 16, 4096, 16, 64
HD = H * D
NC, NS = 2, 16
NW = NC * NS
S2 = S // 2
CH = 16
NCH = S2 // CH
UPW = S // NW
NCH2 = UPW // CH
NBUF = 4

_mesh = plsc.VectorSubcoreMesh(core_axis_name="c", subcore_axis_name="s")


@functools.partial(
    pl.kernel,
    mesh=_mesh,
    out_type=jax.ShapeDtypeStruct((B * S, HD), jnp.float32),
    scratch_types=[
        pltpu.VMEM((NBUF, CH, HD), jnp.float32),
        pltpu.VMEM((CH,), jnp.int32),
        pltpu.SemaphoreType.DMA((NBUF,)),
        pltpu.SemaphoreType.DMA((NBUF,)),
    ],
)
def _sc_body(cache_hbm, update_hbm, idx_hbm, out_hbm, buf, idx_v, isems,
             osems):
    tile = lax.axis_index("s")
    core = lax.axis_index("c")
    base1 = tile * S + core * S2

    def in_copy(c, slot):
        return pltpu.make_async_copy(
            cache_hbm.at[pl.ds(base1 + c * CH, CH)], buf.at[slot],
            isems.at[slot])

    def out_copy(c, slot):
        return pltpu.make_async_copy(
            buf.at[slot], out_hbm.at[pl.ds(base1 + c * CH, CH)],
            osems.at[slot])

    for s in range(NBUF):
        in_copy(s, s).start()

    @pl.loop(0, NCH, step=NBUF)
    def _(c):
        for s in range(NBUF):
            cc = c + s
            in_copy(cc, s).wait()
            out_copy(cc, s).start()
            out_copy(cc, s).wait()
            nxt = cc + NBUF

            @pl.when(nxt < NCH)
            def _():
                in_copy(nxt, s).start()

    plsc.subcore_barrier()

    ub = core * S2 + tile * UPW

    @pl.loop(0, NCH2)
    def _(t):
        off = ub + t * CH
        pltpu.sync_copy(idx_hbm.at[pl.ds(off, CH)], idx_v)
        pltpu.sync_copy(update_hbm.at[pl.ds(off, CH)], buf.at[0])
        pltpu.async_copy(buf.at[0], out_hbm.at[idx_v], osems.at[0]).wait()


@jax.jit
def _dus(cache, update, seq_ids):
    cache2d = cache.reshape(B * S, HD)
    update2d = update.reshape(S, HD)
    idx = seq_ids[0] * S + jnp.arange(S, dtype=jnp.int32)
    out = _sc_body(cache2d, update2d, idx)
    return out.reshape(B, S, H, D)


def kernel(cache, update, seq_ids):
    return _dus(cache, update, seq_ids)


# speedup vs baseline: 2.2676x; 2.2676x over previous
import jax
import jax.numpy as jnp
from jax.experimental import pallas as pl
from jax.experimental.pallas import tpu as pltpu

B, S, H, D = 16, 4096, 16, 64
S_CH = 1024
CPR = S // S_CH
K = B * CPR
NSLOT = 8
LA = 4


def _body(seq_smem, cache_h, update_h, out_h, buf, in_sems, out_sems):
    sid = seq_smem[0]

    def in_copy(j, from_update):
        row, c = divmod(j, CPR)
        src = (update_h.at[0, pl.ds(c * S_CH, S_CH)] if from_update
               else cache_h.at[row, pl.ds(c * S_CH, S_CH)])
        return pltpu.make_async_copy(src, buf.at[j % NSLOT],
                                     in_sems.at[j % NSLOT])

    def out_copy(j):
        row, c = divmod(j, CPR)
        return pltpu.make_async_copy(buf.at[j % NSLOT],
                                     out_h.at[row, pl.ds(c * S_CH, S_CH)],
                                     out_sems.at[j % NSLOT])

    def start_in(j):
        row = j // CPR

        @pl.when(row == sid)
        def _():
            in_copy(j, True).start()

        @pl.when(row != sid)
        def _():
            in_copy(j, False).start()

    for j in range(min(LA, K)):
        start_in(j)
    for k in range(K):
        in_copy(k, False).wait()
        out_copy(k).start()
        nxt = k + LA
        if nxt < K:
            prev = nxt - NSLOT
            if prev >= 0:
                out_copy(prev).wait()
            start_in(nxt)
    for j in range(max(0, K - NSLOT), K):
        out_copy(j).wait()


@jax.jit
def _dus(cache, update, seq_ids):
    return pl.pallas_call(
        _body,
        grid_spec=pltpu.PrefetchScalarGridSpec(
            num_scalar_prefetch=1,
            grid=(),
            in_specs=[
                pl.BlockSpec(memory_space=pl.MemorySpace.ANY),
                pl.BlockSpec(memory_space=pl.MemorySpace.ANY),
            ],
            out_specs=pl.BlockSpec(memory_space=pl.MemorySpace.ANY),
            scratch_shapes=[
                pltpu.VMEM((NSLOT, S_CH, H * D), jnp.float32),
                pltpu.SemaphoreType.DMA((NSLOT,)),
                pltpu.SemaphoreType.DMA((NSLOT,)),
            ],
        ),
        out_shape=jax.ShapeDtypeStruct((B, S, H * D), jnp.float32),
    )(seq_ids, cache, update)


def kernel(cache, update, seq_ids):
    cache3d = cache.reshape(B, S, H * D)
    update3d = update.reshape(1, S, H * D)
    out = _dus(cache3d, update3d, seq_ids)
    return out.reshape(B, S, H, D)
